# Initial kernel scaffold; baseline (speedup 1.0000x reference)
import jax
import jax.numpy as jnp
from jax.experimental import pallas as pl


def kernel(x, edge_index, edge_attr, mask, center, mask_idx, W1, b1, W2, We, lin_w, lin_b):
    raise NotImplementedError("write your pallas kernel here")


# speedup vs baseline: 9.3842x; 9.3842x over previous
import dataclasses
import functools

import jax
import jax.numpy as jnp
from jax import lax
from jax.experimental import pallas as pl
from jax.experimental.pallas import tpu as pltpu
from jax.experimental.pallas import tpu_sc as plsc

N = 10000
E = 320000
D = 128
DE = 16
Q = 512
K_MASK = 4
MAX_NBR = 64
RADIUS = 20.0
THRESHOLD = 0.1

NP = 10240
NPT = 10368
TRASH = 10240
QB = 64
NQB = Q // QB

NSC = 2
NSUB = 16
FH = D // NSC
EPW = E // NSUB
CH = 128
NCHUNK = EPW // CH
CHT = EPW - NCHUNK * CH
RPS = NPT // NSUB
ZR = 72


def _s1_body(xt_ref, cb_ref, xp_ref, w1_ref, w2_ref,
             nin_ref, qany_ref, y_ref, h_ref):
    i = pl.program_id(0)
    xb = xp_ref[...]
    y_ref[...] = jnp.dot(xb, w2_ref[...], preferred_element_type=jnp.float32)
    h_ref[...] = jnp.dot(xb, w1_ref[...], preferred_element_type=jnp.float32)
    xt = xt_ref[...]
    x2 = jnp.sum(xt * xt, axis=0)[None, :]
    col = lax.broadcasted_iota(jnp.int32, (1, NP), 1)
    x2 = jnp.where(col < N, x2, jnp.inf)
    cb = cb_ref[...]
    c2 = jnp.sum(cb * cb, axis=1, keepdims=True)
    cx = lax.dot_general(cb, xt, (((1,), (0,)), ((), ())),
                         preferred_element_type=jnp.float32)
    d2 = jnp.maximum(c2 + x2 - 2.0 * cx, 0.0)
    bits = lax.bitcast_convert_type(d2, jnp.int32)
    ans = jnp.zeros((QB, 1), jnp.int32)
    for b in range(30, -1, -1):
        trial = ans | (1 << b)
        cnt = jnp.sum((bits < trial).astype(jnp.int32), axis=1, keepdims=True)
        ans = jnp.where(cnt <= MAX_NBR - 1, trial, ans)
    kth = lax.bitcast_convert_type(ans, jnp.float32)
    t = jnp.minimum(kth, RADIUS * RADIUS)
    sel = d2 <= t
    nin_new = jnp.any(sel, axis=0)[None, :].astype(jnp.int32)
    qany_ref[0, 0, :] = jnp.any(sel, axis=1).astype(jnp.int32)

    @pl.when(i == 0)
    def _():
        nin_ref[...] = nin_new

    @pl.when(i != 0)
    def _():
        nin_ref[...] = nin_ref[...] | nin_new


def _stage1(xt, center, xp, W1, W2, interpret=False):
    NB = NP // NQB
    return pl.pallas_call(
        _s1_body,
        grid=(NQB,),
        in_specs=[
            pl.BlockSpec((D, NP), lambda i: (0, 0)),
            pl.BlockSpec((QB, D), lambda i: (i, 0)),
            pl.BlockSpec((NB, D), lambda i: (i, 0)),
            pl.BlockSpec((D, D), lambda i: (0, 0)),
            pl.BlockSpec((D, D), lambda i: (0, 0)),
        ],
        out_specs=[
            pl.BlockSpec((1, NP), lambda i: (0, 0)),
            pl.BlockSpec((1, 1, QB), lambda i: (i, 0, 0)),
            pl.BlockSpec((NB, D), lambda i: (i, 0)),
            pl.BlockSpec((NB, D), lambda i: (i, 0)),
        ],
        out_shape=[
            jax.ShapeDtypeStruct((1, NP), jnp.int32),
            jax.ShapeDtypeStruct((NQB, 1, QB), jnp.int32),
            jax.ShapeDtypeStruct((NP, D), jnp.float32),
            jax.ShapeDtypeStruct((NP, D), jnp.float32),
        ],
        interpret=interpret,
    )(xt, center, xp, W1, W2)


def _s2_body(src_hbm, dst_hbm, ea_hbm, nin_hbm, xs_hbm, gx_out, ga_out,
             src_v, dst_v, dstp_v, srcg_v, srct_v, dstt_v, dstpt_v, srcgt_v,
             rows_v, ea_v, rowst_v, eat_v, nin_v, zx_v, za_v,
             gx_sh, ga_sh, sem):
    cid = lax.axis_index("c")
    sid = lax.axis_index("s")
    ebase = sid * EPW

    pltpu.sync_copy(nin_hbm, nin_v)

    @pl.loop(0, ZR)
    def _(r):
        za_v[r, :] = jnp.zeros((DE,), jnp.float32)

        @pl.loop(0, FH, step=16)
        def _(j):
            zx_v[r, pl.ds(j, 16)] = jnp.zeros((16,), jnp.float32)

    r0 = sid * RPS

    @pl.loop(0, RPS, step=ZR)
    def _(k):
        pltpu.sync_copy(zx_v, gx_sh.at[pl.ds(r0 + k, ZR)])
        pltpu.sync_copy(za_v, ga_sh.at[pl.ds(r0 + k, ZR)])

    plsc.subcore_barrier()

    def process(e0, src_b, dst_b, dstp_b, srcg_b, rows_b, ea_b, n):
        pltpu.sync_copy(src_hbm.at[pl.ds(e0, n)], src_b)
        pltpu.sync_copy(dst_hbm.at[pl.ds(e0, n)], dst_b)

        @pl.loop(0, n, step=16)
        def _(j):
            s16 = src_b[pl.ds(j, 16)]
            d16 = dst_b[pl.ds(j, 16)]
            ns = plsc.load_gather(nin_v, [s16])
            nd = plsc.load_gather(nin_v, [d16])
            ok = (ns > 0) & (nd > 0)
            srcg_b[pl.ds(j, 16)] = s16 + cid * N
            trash = TRASH + lax.iota(jnp.int32, 16) + j
            dstp_b[pl.ds(j, 16)] = jnp.where(ok, d16, trash)

        cp = pltpu.async_copy(xs_hbm.at[srcg_b], rows_b, sem)

        @pl.when(cid == 0)
        def _():
            pltpu.sync_copy(ea_hbm.at[pl.ds(e0, n)], ea_b)
            pltpu.sync_copy(ea_b, ga_sh.at[dstp_b], add=True)

        cp.wait()
        pltpu.sync_copy(rows_b, gx_sh.at[dstp_b], add=True)

    @pl.loop(0, NCHUNK)
    def _(ci):
        process(ebase + ci * CH, src_v, dst_v, dstp_v, srcg_v,
                rows_v, ea_v, CH)

    process(ebase + NCHUNK * CH, srct_v, dstt_v, dstpt_v, srcgt_v,
            rowst_v, eat_v, CHT)

    plsc.subcore_barrier()
    pltpu.sync_copy(gx_sh.at[pl.ds(r0, RPS)], gx_out.at[cid, pl.ds(r0, RPS)])

    @pl.when(cid == 0)
    def _():
        pltpu.sync_copy(ga_sh.at[pl.ds(r0, RPS)], ga_out.at[pl.ds(r0, RPS)])


def _stage2(src, dst, edge_attr, nin_flat, xsplit):
    mesh = plsc.VectorSubcoreMesh(core_axis_name="c", subcore_axis_name="s")
    cp = pltpu.CompilerParams()
    for fld, val in (("needs_layout_passes", False),
                     ("use_tc_tiling_on_sc", False)):
        if fld in pltpu.CompilerParams.__dataclass_fields__:
            cp = dataclasses.replace(cp, **{fld: val})
    f = pl.kernel(
        _s2_body,
        mesh=mesh,
        compiler_params=cp,
        out_type=(
            jax.ShapeDtypeStruct((NSC, NPT, FH), jnp.float32),
            jax.ShapeDtypeStruct((NPT, DE), jnp.float32),
        ),
        scratch_types=[
            pltpu.VMEM((CH,), jnp.int32),
            pltpu.VMEM((CH,), jnp.int32),
            pltpu.VMEM((CH,), jnp.int32),
            pltpu.VMEM((CH,), jnp.int32),
            pltpu.VMEM((CHT,), jnp.int32),
            pltpu.VMEM((CHT,), jnp.int32),
            pltpu.VMEM((CHT,), jnp.int32),
            pltpu.VMEM((CHT,), jnp.int32),
            pltpu.VMEM((CH, FH), jnp.float32),
            pltpu.VMEM((CH, DE), jnp.float32),
            pltpu.VMEM((CHT, FH), jnp.float32),
            pltpu.VMEM((CHT, DE), jnp.float32),
            pltpu.VMEM((NP,), jnp.int32),
            pltpu.VMEM((ZR, FH), jnp.float32),
            pltpu.VMEM((ZR, DE), jnp.float32),
            pltpu.VMEM_SHARED((NPT, FH), jnp.float32),
            pltpu.VMEM_SHARED((NPT, DE), jnp.float32),
            pltpu.SemaphoreType.DMA,
        ],
    )
    return f(src, dst, edge_attr, nin_flat, xsplit)


def _s3_body(h_ref, gy_ref, ga_ref, nin_ref, mem_ref,
             b1_ref, we_ref, lw_ref, lb_ref, o_ref):
    gyb = jnp.concatenate([gy_ref[0], gy_ref[1]], axis=1)
    gab = ga_ref[...]
    web = we_ref[...]
    agge = gab[:, 0:1] * web[0:1, :]
    for k in range(1, DE):
        agge = agge + gab[:, k:k + 1] * web[k:k + 1, :]
    wt = jnp.maximum((h_ref[...] + b1_ref[...]) + (gyb + agge), 0.0)
    wtb = wt.astype(jnp.bfloat16).astype(jnp.float32)
    wv = jnp.maximum(
        jnp.dot(wtb, lw_ref[...], preferred_element_type=jnp.float32)
        + lb_ref[...], 0.0)
    memb = (nin_ref[...] + mem_ref[...]) > 0
    o_ref[...] = (memb & (wv > THRESHOLD)).astype(jnp.int32)


def _stage3(h, gy2, ga2, nin_col, mem_col, b1, We, lin_w, lin_b,
            interpret=False):
    BN = 1024
    g = NP // BN
    return pl.pallas_call(
        _s3_body,
        grid=(g,),
        in_specs=[
            pl.BlockSpec((BN, D), lambda i: (i, 0)),
            pl.BlockSpec((NSC, BN, FH), lambda i: (0, i, 0)),
            pl.BlockSpec((BN, DE), lambda i: (i, 0)),
            pl.BlockSpec((BN, 1), lambda i: (i, 0)),
            pl.BlockSpec((BN, 1), lambda i: (i, 0)),
            pl.BlockSpec((1, D), lambda i: (0, 0)),
            pl.BlockSpec((DE, D), lambda i: (0, 0)),
            pl.BlockSpec((D, 1), lambda i: (0, 0)),
            pl.BlockSpec((1, 1), lambda i: (0, 0)),
        ],
        out_specs=pl.BlockSpec((BN, 1), lambda i: (i, 0)),
        out_shape=jax.ShapeDtypeStruct((NP, 1), jnp.int32),
        interpret=interpret,
    )(h, gy2, ga2, nin_col, mem_col, b1, We, lin_w, lin_b)


def _b16(a):
    return a.astype(jnp.bfloat16).astype(jnp.float32)


def kernel(x, edge_index, edge_attr, mask, center, mask_idx,
           W1, b1, W2, We, lin_w, lin_b):
    xp = jnp.pad(x, ((0, NP - N), (0, 0)))
    xt = xp.T
    nin2d, qany3d, Y, H = _stage1(xt, center, xp, W1, W2)
    nin_flat = nin2d.reshape(NP)
    src = edge_index[0].astype(jnp.int32)
    dst = edge_index[1].astype(jnp.int32)
    ysplit = jnp.concatenate([Y[:N, :FH], Y[:N, FH:]], axis=0)
    gy2, ga2 = _stage2(src, dst, _b16(edge_attr), nin_flat, ysplit)
    mem_extra = jnp.pad(qany3d.reshape(Q), (0, NP - Q))
    out_col = _stage3(H, gy2, ga2, nin_flat[:, None],
                      mem_extra[:, None], b1.reshape(1, D),
                      _b16(We), _b16(lin_w), lin_b.reshape(1, 1))
    return out_col[:N, 0].astype(bool)
